# baseline (speedup 1.0000x reference)
import functools

import jax
import jax.numpy as jnp
from jax import lax
from jax.experimental import pallas as pl
from jax.experimental.pallas import tpu as pltpu
from jax.experimental.pallas import tpu_sc as plsc

N = 10000
E = 320000
D = 128
NP = 10240
NSUB = 16
EPW = E // NSUB
B = 48
EP = 321024
ROWS_PER_SUB = NP // NSUB


@functools.cache
def _make_sc_agg(split):
    mesh = plsc.VectorSubcoreMesh(core_axis_name="c", subcore_axis_name="s",
                                  num_cores=2, num_subcores=NSUB)
    dh = 128
    fchunks = dh // 16
    epw = EP // NSUB if split == "feature" else EP // NSUB // 2
    nb = epw // B

    @functools.partial(
        pl.kernel,
        out_type=jax.ShapeDtypeStruct((2, NP, dh), jnp.float32),
        mesh=mesh,
        scratch_types=[
            pltpu.VMEM((B,), jnp.int32),
            pltpu.VMEM((B,), jnp.int32),
            pltpu.VMEM((B,), jnp.int32),
            pltpu.VMEM((B,), jnp.int32),
            pltpu.VMEM((B, dh), jnp.float32),
            pltpu.VMEM((B, dh), jnp.float32),
            pltpu.VMEM((B, dh), jnp.float32),
            pltpu.VMEM((128, dh), jnp.float32),
            pltpu.VMEM_SHARED((NP, dh), jnp.float32),
            pltpu.SemaphoreType.DMA,
        ],
    )
    def sc_agg(kneg_hbm, q_hbm, v_hbm, src_hbm, dst_hbm, out_hbm,
               srcb, dstb, kidx, qidx, kbuf, qbuf, vbuf, zbuf, acc, sem):
        c = lax.axis_index("c")
        s = lax.axis_index("s")

        def zrow(i, carry):
            for t in range(fchunks):
                zbuf[i, pl.ds(16 * t, 16)] = jnp.zeros((16,), jnp.float32)
            return carry
        lax.fori_loop(0, 128, zrow, 0)
        r0 = s * ROWS_PER_SUB
        for t in range(ROWS_PER_SUB // 128):
            pltpu.sync_copy(zbuf, acc.at[pl.ds(r0 + 128 * t, 128)])
        plsc.subcore_barrier()

        def batch(j, carry):
            if split == "feature":
                base = s * epw + j * B
            else:
                base = c * (EP // 2) + s * epw + j * B
            pltpu.sync_copy(src_hbm.at[pl.ds(base, B)], srcb)
            pltpu.sync_copy(dst_hbm.at[pl.ds(base, B)], dstb)
            for t in range(B // 16):
                sl = pl.ds(16 * t, 16)
                dvc = jnp.minimum(dstb[sl], N - 1)
                if split == "feature":
                    kidx[sl] = dvc * 2 + c
                    qidx[sl] = srcb[sl] * 2 + c
                else:
                    kidx[sl] = dvc
                    qidx[sl] = srcb[sl]
            d1 = pltpu.async_copy(kneg_hbm.at[kidx], kbuf, sem)
            d2 = pltpu.async_copy(q_hbm.at[qidx], qbuf, sem)
            d3 = pltpu.async_copy(v_hbm.at[qidx], vbuf, sem)
            d1.wait()
            d2.wait()
            d3.wait()

            def row(i, carry2):
                for t in range(fchunks):
                    sl = pl.ds(16 * t, 16)
                    e = jnp.exp(kbuf[i, sl] - qbuf[i, sl])
                    vbuf[i, sl] = vbuf[i, sl] / (e + 1.0)
                return carry2
            lax.fori_loop(0, B, row, 0)
            pltpu.sync_copy(vbuf, acc.at[dstb], add=True)
            return carry
        lax.fori_loop(0, nb, batch, 0)

        plsc.subcore_barrier()
        for t in range(ROWS_PER_SUB // 128):
            rr = r0 + 128 * t
            pltpu.sync_copy(acc.at[pl.ds(rr, 128)], out_hbm.at[c, pl.ds(rr, 128)])

    return sc_agg


_BN = 1000


def _tc_proj_kernel(x_ref, w_ref, b_ref, kneg_ref, q_ref, v_ref, s_ref, *, dout):
    x = x_ref[...]
    w = w_ref[...]
    b = b_ref[...]
    out = jnp.dot(x, w, preferred_element_type=jnp.float32) + b
    kneg_ref[...] = -out[:, :dout]
    q_ref[...] = out[:, dout:2 * dout]
    v_ref[...] = out[:, 2 * dout:3 * dout]
    s_ref[...] = out[:, 3 * dout:]


def _tc_proj(x, wcat, bcat, dout):
    din = x.shape[1]
    n = x.shape[0]
    grid = n // _BN
    shp = jax.ShapeDtypeStruct((n, dout), jnp.float32)
    return pl.pallas_call(
        functools.partial(_tc_proj_kernel, dout=dout),
        grid=(grid,),
        in_specs=[
            pl.BlockSpec((_BN, din), lambda i: (i, 0)),
            pl.BlockSpec((din, 4 * dout), lambda i: (0, 0)),
            pl.BlockSpec((1, 4 * dout), lambda i: (0, 0)),
        ],
        out_specs=[pl.BlockSpec((_BN, dout), lambda i: (i, 0))] * 4,
        out_shape=[shp, shp, shp, shp],
    )(x, wcat, bcat)


def _tc_combine_proj_kernel(agg_ref, s_ref, w_ref, b_ref,
                            kneg_ref, q_ref, v_ref, out_s_ref, *, dout):
    h = jnp.concatenate([agg_ref[0], agg_ref[1]], axis=1) + s_ref[...]
    h = jnp.maximum(h, 0.0)
    out = jnp.dot(h, w_ref[...], preferred_element_type=jnp.float32) + b_ref[...]
    kneg_ref[...] = -out[:, :dout]
    q_ref[...] = out[:, dout:2 * dout]
    v_ref[...] = out[:, 2 * dout:3 * dout]
    out_s_ref[...] = out[:, 3 * dout:]


def _tc_combine_proj(agg, sres, wcat, bcat, dout):
    dh = agg.shape[2]
    din = 2 * dh
    grid = N // _BN
    shp = jax.ShapeDtypeStruct((N, dout), jnp.float32)
    return pl.pallas_call(
        functools.partial(_tc_combine_proj_kernel, dout=dout),
        grid=(grid,),
        in_specs=[
            pl.BlockSpec((2, _BN, dh), lambda i: (0, i, 0)),
            pl.BlockSpec((_BN, din), lambda i: (i, 0)),
            pl.BlockSpec((din, 4 * dout), lambda i: (0, 0)),
            pl.BlockSpec((1, 4 * dout), lambda i: (0, 0)),
        ],
        out_specs=[pl.BlockSpec((_BN, dout), lambda i: (i, 0))] * 4,
        out_shape=[shp, shp, shp, shp],
    )(agg, sres, wcat, bcat)


def _tc_head_kernel(agg_ref, s_ref, w1_ref, b1_ref, w2_ref, b2_ref, o_ref):
    h = agg_ref[0] + agg_ref[1] + s_ref[...]
    h = jnp.maximum(h, 0.0)
    m = jnp.dot(h, w1_ref[...], preferred_element_type=jnp.float32) + b1_ref[...]
    m = jnp.maximum(m, 0.0)
    o = jnp.dot(m, w2_ref[...], preferred_element_type=jnp.float32) + b2_ref[...]
    o_ref[...] = jnp.maximum(o, 0.0)


def _tc_head(agg, sres, w1, b1, w2p, b2p):
    dh = agg.shape[2]
    grid = N // _BN
    return pl.pallas_call(
        _tc_head_kernel,
        grid=(grid,),
        in_specs=[
            pl.BlockSpec((2, _BN, dh), lambda i: (0, i, 0)),
            pl.BlockSpec((_BN, dh), lambda i: (i, 0)),
            pl.BlockSpec((128, 64), lambda i: (0, 0)),
            pl.BlockSpec((1, 64), lambda i: (0, 0)),
            pl.BlockSpec((64, 128), lambda i: (0, 0)),
            pl.BlockSpec((1, 128), lambda i: (0, 0)),
        ],
        out_specs=pl.BlockSpec((_BN, 128), lambda i: (i, 0)),
        out_shape=jax.ShapeDtypeStruct((N, 128), jnp.float32),
    )(agg, sres, w1, b1, w2p, b2p)


def kernel(x, edge_index, Wk1, bk1, Wq1, bq1, Wv1, bv1, Ws1, b1,
           Wk2, bk2, Wq2, bq2, Wv2, bv2, Ws2, b2, Wm1, bm1, Wm2, bm2):
    pad = EP - E
    src = jnp.concatenate([edge_index[0], jnp.zeros((pad,), jnp.int32)])
    dst = jnp.concatenate([edge_index[1], jnp.full((pad,), N, jnp.int32)])

    w1cat = jnp.concatenate([Wk1, Wq1, Wv1, Ws1], axis=1)
    b1cat = jnp.concatenate([bk1, bq1, bv1, b1])[None, :]
    w2cat = jnp.concatenate([Wk2, Wq2, Wv2, Ws2], axis=1)
    b2cat = jnp.concatenate([bk2, bq2, bv2, b2])[None, :]
    wm2p = jnp.pad(Wm2, ((0, 0), (0, 125)))
    bm2p = jnp.pad(bm2, (0, 125))[None, :]

    kneg1, q1, v1, s1 = _tc_proj(x, w1cat, b1cat, 256)
    agg1 = _make_sc_agg("feature")(
        kneg1.reshape(2 * N, 128), q1.reshape(2 * N, 128),
        v1.reshape(2 * N, 128), src, dst)
    kneg2, q2, v2, s2 = _tc_combine_proj(agg1, s1, w2cat, b2cat, 128)
    agg2 = _make_sc_agg("edge")(kneg2, q2, v2, src, dst)
    out = _tc_head(agg2, s2, Wm1, bm1[None, :], wm2p, bm2p)
    return out[:, :3]


# speedup vs baseline: 2.2258x; 2.2258x over previous
import functools

import jax
import jax.numpy as jnp
from jax import lax
from jax.experimental import pallas as pl
from jax.experimental.pallas import tpu as pltpu
from jax.experimental.pallas import tpu_sc as plsc

N = 10000
E = 320000
D = 128
NP = 10240
NSUB = 16
EPW = E // NSUB
B = 48
EP = 321024
ROWS_PER_SUB = NP // NSUB


@functools.cache
def _make_sc_agg(split):
    mesh = plsc.VectorSubcoreMesh(core_axis_name="c", subcore_axis_name="s",
                                  num_cores=2, num_subcores=NSUB)
    dh = 128
    fchunks = dh // 16
    epw = EP // NSUB if split == "feature" else EP // NSUB // 2
    nb = epw // B
    npairs = (nb - 2) // 2
    leftover = nb - 2 * npairs

    @functools.partial(
        pl.kernel,
        out_type=jax.ShapeDtypeStruct((2, NP, dh), jnp.float32),
        mesh=mesh,
        scratch_types=[
            pltpu.VMEM((B,), jnp.int32),
            pltpu.VMEM((B,), jnp.int32),
            pltpu.VMEM((B,), jnp.int32),
            pltpu.VMEM((B,), jnp.int32),
            pltpu.VMEM((B,), jnp.int32),
            pltpu.VMEM((B,), jnp.int32),
            pltpu.VMEM((B,), jnp.int32),
            pltpu.VMEM((B,), jnp.int32),
            pltpu.VMEM((B,), jnp.int32),
            pltpu.VMEM((B,), jnp.int32),
            pltpu.VMEM((B, dh), jnp.float32),
            pltpu.VMEM((B, dh), jnp.float32),
            pltpu.VMEM((B, dh), jnp.float32),
            pltpu.VMEM((B, dh), jnp.float32),
            pltpu.VMEM((B, dh), jnp.float32),
            pltpu.VMEM((B, dh), jnp.float32),
            pltpu.VMEM((64, dh), jnp.float32),
            pltpu.VMEM_SHARED((NP, dh), jnp.float32),
            pltpu.SemaphoreType.DMA,
            pltpu.SemaphoreType.DMA,
            pltpu.SemaphoreType.DMA,
            pltpu.SemaphoreType.DMA,
        ],
    )
    def sc_agg(kneg_hbm, q_hbm, v_hbm, src_hbm, dst_hbm, out_hbm,
               srcb0, srcb1, dstb0, dstb1, kidx0, kidx1, qidx0, qidx1,
               sdst0, sdst1, kbuf0, kbuf1, qbuf0, qbuf1, vbuf0, vbuf1,
               zbuf, acc, semi0, semi1, semg0, semg1):
        srcb = (srcb0, srcb1)
        dstb = (dstb0, dstb1)
        kidx = (kidx0, kidx1)
        qidx = (qidx0, qidx1)
        sdst = (sdst0, sdst1)
        kbuf = (kbuf0, kbuf1)
        qbuf = (qbuf0, qbuf1)
        vbuf = (vbuf0, vbuf1)
        semi = (semi0, semi1)
        semg = (semg0, semg1)
        c = lax.axis_index("c")
        s = lax.axis_index("s")
        base0 = s * epw if split == "feature" else c * (EP // 2) + s * epw

        def zrow(i, carry):
            for t in range(fchunks):
                zbuf[i, pl.ds(16 * t, 16)] = jnp.zeros((16,), jnp.float32)
            return carry
        lax.fori_loop(0, 64, zrow, 0)
        r0 = s * ROWS_PER_SUB
        for t in range(ROWS_PER_SUB // 64):
            pltpu.sync_copy(zbuf, acc.at[pl.ds(r0 + 64 * t, 64)])
        plsc.subcore_barrier()

        def issue_idx(j, slot):
            base = base0 + j * B
            pltpu.async_copy(src_hbm.at[pl.ds(base, B)], srcb[slot], semi[slot])
            pltpu.async_copy(dst_hbm.at[pl.ds(base, B)], dstb[slot], semi[slot])

        def drain_idx(slot):
            dummy = src_hbm.at[pl.ds(0, B)]
            pltpu.make_async_copy(dummy, srcb[slot], semi[slot]).wait()
            pltpu.make_async_copy(dummy, dstb[slot], semi[slot]).wait()

        def transform_fire(slot):
            for t in range(B // 16):
                sl = pl.ds(16 * t, 16)
                sv = srcb[slot][sl]
                dv = dstb[slot][sl]
                dvc = jnp.minimum(dv, N - 1)
                if split == "feature":
                    kidx[slot][sl] = dvc * 2 + c
                    qidx[slot][sl] = sv * 2 + c
                else:
                    kidx[slot][sl] = dvc
                    qidx[slot][sl] = sv
                sdst[slot][sl] = dv
            pltpu.async_copy(kneg_hbm.at[kidx[slot]], kbuf[slot], semg[slot])
            pltpu.async_copy(q_hbm.at[qidx[slot]], qbuf[slot], semg[slot])
            pltpu.async_copy(v_hbm.at[qidx[slot]], vbuf[slot], semg[slot])

        def drain_gathers(slot):
            pltpu.make_async_copy(kneg_hbm.at[kidx[slot]], kbuf[slot],
                                  semg[slot]).wait()
            pltpu.make_async_copy(q_hbm.at[qidx[slot]], qbuf[slot],
                                  semg[slot]).wait()
            pltpu.make_async_copy(v_hbm.at[qidx[slot]], vbuf[slot],
                                  semg[slot]).wait()

        def process(slot):
            kb, qb, vb = kbuf[slot], qbuf[slot], vbuf[slot]

            def row(i, carry2):
                for t in range(fchunks):
                    sl = pl.ds(16 * t, 16)
                    e = jnp.exp(kb[i, sl] - qb[i, sl])
                    vb[i, sl] = vb[i, sl] / (e + 1.0)
                return carry2
            lax.fori_loop(0, B, row, 0)
            pltpu.sync_copy(vb, acc.at[sdst[slot]], add=True)

        issue_idx(0, 0)
        drain_idx(0)
        transform_fire(0)
        issue_idx(1, 1)

        def body(jj, carry):
            j0 = 2 * jj
            drain_idx(1)
            transform_fire(1)
            issue_idx(j0 + 2, 0)
            drain_gathers(0)
            process(0)
            drain_idx(0)
            transform_fire(0)
            issue_idx(j0 + 3, 1)
            drain_gathers(1)
            process(1)
            return carry
        lax.fori_loop(0, npairs, body, 0)

        drain_idx(1)
        transform_fire(1)
        if leftover == 3:
            issue_idx(nb - 1, 0)
        drain_gathers(0)
        process(0)
        if leftover == 3:
            drain_idx(0)
            transform_fire(0)
        drain_gathers(1)
        process(1)
        if leftover == 3:
            drain_gathers(0)
            process(0)

        plsc.subcore_barrier()
        for t in range(ROWS_PER_SUB // 64):
            rr = r0 + 64 * t
            pltpu.sync_copy(acc.at[pl.ds(rr, 64)], out_hbm.at[c, pl.ds(rr, 64)])

    return sc_agg


_BN = 1000


def _tc_proj_kernel(x_ref, w_ref, b_ref, kneg_ref, q_ref, v_ref, s_ref, *, dout):
    x = x_ref[...]
    w = w_ref[...]
    b = b_ref[...]
    out = jnp.dot(x, w, preferred_element_type=jnp.float32) + b
    kneg_ref[...] = -out[:, :dout]
    q_ref[...] = out[:, dout:2 * dout]
    v_ref[...] = out[:, 2 * dout:3 * dout]
    s_ref[...] = out[:, 3 * dout:]


def _tc_proj(x, wcat, bcat, dout):
    din = x.shape[1]
    n = x.shape[0]
    grid = n // _BN
    shp = jax.ShapeDtypeStruct((n, dout), jnp.float32)
    return pl.pallas_call(
        functools.partial(_tc_proj_kernel, dout=dout),
        grid=(grid,),
        in_specs=[
            pl.BlockSpec((_BN, din), lambda i: (i, 0)),
            pl.BlockSpec((din, 4 * dout), lambda i: (0, 0)),
            pl.BlockSpec((1, 4 * dout), lambda i: (0, 0)),
        ],
        out_specs=[pl.BlockSpec((_BN, dout), lambda i: (i, 0))] * 4,
        out_shape=[shp, shp, shp, shp],
    )(x, wcat, bcat)


def _tc_combine_proj_kernel(agg_ref, s_ref, w_ref, b_ref,
                            kneg_ref, q_ref, v_ref, out_s_ref, *, dout):
    h = jnp.concatenate([agg_ref[0], agg_ref[1]], axis=1) + s_ref[...]
    h = jnp.maximum(h, 0.0)
    out = jnp.dot(h, w_ref[...], preferred_element_type=jnp.float32) + b_ref[...]
    kneg_ref[...] = -out[:, :dout]
    q_ref[...] = out[:, dout:2 * dout]
    v_ref[...] = out[:, 2 * dout:3 * dout]
    out_s_ref[...] = out[:, 3 * dout:]


def _tc_combine_proj(agg, sres, wcat, bcat, dout):
    dh = agg.shape[2]
    din = 2 * dh
    grid = N // _BN
    shp = jax.ShapeDtypeStruct((N, dout), jnp.float32)
    return pl.pallas_call(
        functools.partial(_tc_combine_proj_kernel, dout=dout),
        grid=(grid,),
        in_specs=[
            pl.BlockSpec((2, _BN, dh), lambda i: (0, i, 0)),
            pl.BlockSpec((_BN, din), lambda i: (i, 0)),
            pl.BlockSpec((din, 4 * dout), lambda i: (0, 0)),
            pl.BlockSpec((1, 4 * dout), lambda i: (0, 0)),
        ],
        out_specs=[pl.BlockSpec((_BN, dout), lambda i: (i, 0))] * 4,
        out_shape=[shp, shp, shp, shp],
    )(agg, sres, wcat, bcat)


def _tc_head_kernel(agg_ref, s_ref, w1_ref, b1_ref, w2_ref, b2_ref, o_ref):
    h = agg_ref[0] + agg_ref[1] + s_ref[...]
    h = jnp.maximum(h, 0.0)
    m = jnp.dot(h, w1_ref[...], preferred_element_type=jnp.float32) + b1_ref[...]
    m = jnp.maximum(m, 0.0)
    o = jnp.dot(m, w2_ref[...], preferred_element_type=jnp.float32) + b2_ref[...]
    o_ref[...] = jnp.maximum(o, 0.0)


def _tc_head(agg, sres, w1, b1, w2p, b2p):
    dh = agg.shape[2]
    grid = N // _BN
    return pl.pallas_call(
        _tc_head_kernel,
        grid=(grid,),
        in_specs=[
            pl.BlockSpec((2, _BN, dh), lambda i: (0, i, 0)),
            pl.BlockSpec((_BN, dh), lambda i: (i, 0)),
            pl.BlockSpec((128, 64), lambda i: (0, 0)),
            pl.BlockSpec((1, 64), lambda i: (0, 0)),
            pl.BlockSpec((64, 128), lambda i: (0, 0)),
            pl.BlockSpec((1, 128), lambda i: (0, 0)),
        ],
        out_specs=pl.BlockSpec((_BN, 128), lambda i: (i, 0)),
        out_shape=jax.ShapeDtypeStruct((N, 128), jnp.float32),
    )(agg, sres, w1, b1, w2p, b2p)


def kernel(x, edge_index, Wk1, bk1, Wq1, bq1, Wv1, bv1, Ws1, b1,
           Wk2, bk2, Wq2, bq2, Wv2, bv2, Ws2, b2, Wm1, bm1, Wm2, bm2):
    pad = EP - E
    src = jnp.concatenate([edge_index[0], jnp.zeros((pad,), jnp.int32)])
    dst = jnp.concatenate([edge_index[1], jnp.full((pad,), N, jnp.int32)])

    w1cat = jnp.concatenate([Wk1, Wq1, Wv1, Ws1], axis=1)
    b1cat = jnp.concatenate([bk1, bq1, bv1, b1])[None, :]
    w2cat = jnp.concatenate([Wk2, Wq2, Wv2, Ws2], axis=1)
    b2cat = jnp.concatenate([bk2, bq2, bv2, b2])[None, :]
    wm2p = jnp.pad(Wm2, ((0, 0), (0, 125)))
    bm2p = jnp.pad(bm2, (0, 125))[None, :]

    kneg1, q1, v1, s1 = _tc_proj(x, w1cat, b1cat, 256)
    agg1 = _make_sc_agg("feature")(
        kneg1.reshape(2 * N, 128), q1.reshape(2 * N, 128),
        v1.reshape(2 * N, 128), src, dst)
    kneg2, q2, v2, s2 = _tc_combine_proj(agg1, s1, w2cat, b2cat, 128)
    agg2 = _make_sc_agg("edge")(kneg2, q2, v2, src, dst)
    out = _tc_head(agg2, s2, Wm1, bm1[None, :], wm2p, bm2p)
    return out[:, :3]
